# baseline (speedup 1.0000x reference)
import functools

import jax
import jax.numpy as jnp
from jax import lax
from jax.experimental import pallas as pl
from jax.experimental.pallas import tpu as pltpu
from jax.experimental.pallas import tpu_sc as plsc

_K = 4
_TOP_M = 8
_TEMP = 0.07
_EPS = 1e-06
_NUM_CLASSES = 1000
_TB = 4096

_NW = 32
_WPK = _NW // _K
_RPW = _NUM_CLASSES // _WPK


def _wu_sc_kernel(clsw_ref, u_ref, out_ref, w_vm, u_vm, o_vm):
    D = 1024
    wid = lax.axis_index("s") * 2 + lax.axis_index("c")
    k = wid // _WPK
    j = wid % _WPK
    pltpu.sync_copy(u_ref.at[pl.ds(k * D, D)], u_vm)
    src = (k * _NUM_CLASSES + j * _RPW) * D
    pltpu.sync_copy(clsw_ref.at[pl.ds(src, _RPW * D)], w_vm)

    for g in range(8):
        rows = 16 if g < 7 else _RPW - 7 * 16
        zero = jnp.zeros((16,), jnp.float32)

        def body(c, accs):
            uc = u_vm[pl.ds(c * 16, 16)]
            return tuple(
                accs[r] + w_vm[pl.ds((g * 16 + r) * D + c * 16, 16)] * uc
                for r in range(rows))

        accs = lax.fori_loop(0, D // 16, body, tuple(zero for _ in range(rows)))
        for r in range(rows):
            o_vm[pl.ds(r * 16, 16)] = accs[r]
        for r in range(rows, 16):
            o_vm[pl.ds(r * 16, 16)] = zero
        pltpu.sync_copy(o_vm, out_ref.at[wid, g])


def _stats_for_batch(b, u, beta_row, coeff_ref, out_ref, *, K, T, top_m):
    plane = coeff_ref[b]
    gram = jax.lax.dot_general(
        u, u, (((1,), (1,)), ((), ())),
        preferred_element_type=jnp.float32,
        precision=jax.lax.Precision.HIGHEST,
    )
    row_i = jax.lax.broadcasted_iota(jnp.int32, (K, K), 0)
    col_j = jax.lax.broadcasted_iota(jnp.int32, (K, K), 1)
    lower = (col_j < row_i).astype(jnp.float32)
    L = lower * beta_row * gram
    eye = (col_j == row_i).astype(jnp.float32)
    hp = jax.lax.Precision.HIGHEST
    L2 = jax.lax.dot_general(L, L, (((1,), (0,)), ((), ())),
                             preferred_element_type=jnp.float32, precision=hp)
    L3 = jax.lax.dot_general(L2, L, (((1,), (0,)), ((), ())),
                             preferred_element_type=jnp.float32, precision=hp)
    M = eye - L + L2 - L3

    coeff = jnp.zeros_like(plane)
    for j in range(K):
        coeff = coeff + M[:, j:j + 1] * plane[j:j + 1, :]

    m = jnp.max(coeff, axis=1, keepdims=True)
    e = jnp.exp((coeff - m) * (1.0 / _TEMP))
    z = jnp.sum(e, axis=1, keepdims=True)
    s_soft = jnp.sum(e * coeff, axis=1, keepdims=True) / z

    iota = jax.lax.broadcasted_iota(jnp.int32, (K, T), 1)
    cur = coeff
    s_top = jnp.zeros((K, 1), dtype=jnp.float32)
    for _ in range(top_m):
        mx = jnp.max(cur, axis=1, keepdims=True)
        s_top = s_top + mx
        hit = jnp.where(cur == mx, iota, T)
        first = jnp.min(hit, axis=1, keepdims=True)
        cur = jnp.where(iota == first, jnp.float32(-jnp.inf), cur)

    out_ref[:, b:b + 1] = 0.5 * (s_top + s_soft)


def _stream_kernel(e_ref, u_ref, beta_ref, out_ref, coeff_ref,
                   *, B, T, K, top_m):
    i = pl.program_id(0)
    tblks = T // _TB
    a_steps = B * tblks
    u = u_ref[...]

    @pl.when(i < a_steps)
    def _():
        res = jax.lax.dot_general(
            u, e_ref[0], (((1,), (1,)), ((), ())),
            preferred_element_type=jnp.float32,
        )
        for s in range(a_steps):
            @pl.when(i == s)
            def _():
                b, tb = divmod(s, tblks)
                coeff_ref[b, :, tb * _TB:(tb + 1) * _TB] = res

    for b in range(B):
        @pl.when(i == (b + 1) * tblks)
        def _():
            _stats_for_batch(b, u, beta_ref[...], coeff_ref, out_ref,
                             K=K, T=T, top_m=top_m)


def _assemble_kernel(c_ref, wup_ref, clsb_ref, alpha_ref, out_ref):
    hp = jax.lax.Precision.HIGHEST
    K = c_ref.shape[0]
    wu = jnp.sum(wup_ref[...], axis=2)
    ac = alpha_ref[...].reshape(K, 1) * c_ref[...]
    logits = jax.lax.dot_general(
        ac, wu, (((0,), (0,)), ((), ())),
        preferred_element_type=jnp.float32, precision=hp,
    )
    bias = jax.lax.dot_general(
        alpha_ref[...], clsb_ref[...], (((1,), (0,)), ((), ())),
        preferred_element_type=jnp.float32, precision=hp,
    )
    out_ref[...] = logits + bias


def kernel(E, v, m_logits, cls_W, cls_b, beta, alpha):
    B, T, D = E.shape
    K = v.shape[0]
    top_m = min(_TOP_M, T)
    tblks = T // _TB
    a_steps = B * tblks

    mk = jax.nn.sigmoid(m_logits)
    vk = v * mk
    U = vk / (jnp.linalg.norm(vk, axis=1, keepdims=True) + _EPS)

    mesh = plsc.VectorSubcoreMesh(core_axis_name="c", subcore_axis_name="s")
    wu_sc = functools.partial(
        pl.kernel, mesh=mesh,
        out_type=jax.ShapeDtypeStruct((_NW, 8, 256), jnp.float32),
        scratch_types=[
            pltpu.VMEM((_RPW * D,), jnp.float32),
            pltpu.VMEM((D,), jnp.float32),
            pltpu.VMEM((256,), jnp.float32),
        ],
    )(_wu_sc_kernel)
    wu_pad = wu_sc(cls_W.reshape(-1), U.reshape(-1))
    wup = wu_pad.reshape(_NW, 128, 16)[:, :_RPW, :].reshape(
        K, _NUM_CLASSES, 16)

    stream = functools.partial(_stream_kernel, B=B, T=T, K=K, top_m=top_m)
    c = pl.pallas_call(
        stream,
        grid=(a_steps + 1,),
        in_specs=[
            pl.BlockSpec(
                (1, _TB, D),
                lambda i: (jnp.minimum(i, a_steps - 1) // tblks,
                           jnp.minimum(i, a_steps - 1) % tblks, 0)),
            pl.BlockSpec((K, D), lambda i: (0, 0)),
            pl.BlockSpec((1, K), lambda i: (0, 0)),
        ],
        out_specs=pl.BlockSpec((K, B), lambda i: (0, 0)),
        out_shape=jax.ShapeDtypeStruct((K, B), jnp.float32),
        scratch_shapes=[pltpu.VMEM((B, K, T), jnp.float32)],
    )(E, U, beta.reshape(1, K))

    logits = pl.pallas_call(
        _assemble_kernel,
        out_shape=jax.ShapeDtypeStruct((B, _NUM_CLASSES), jnp.float32),
    )(c, wup, cls_b, alpha.reshape(1, K))
    return logits


# speedup vs baseline: 1.6043x; 1.6043x over previous
import functools

import jax
import jax.numpy as jnp
from jax.experimental import pallas as pl
from jax.experimental.pallas import tpu as pltpu

_K = 4
_TOP_M = 8
_TEMP = 0.07
_EPS = 1e-06
_NUM_CLASSES = 1000
_TB = 4096


def _stats_all(u, beta_row, coeff_ref, *, B, K, T, top_m):
    c0 = coeff_ref[...]
    gram = jax.lax.dot_general(
        u, u, (((1,), (1,)), ((), ())),
        preferred_element_type=jnp.float32,
        precision=jax.lax.Precision.HIGHEST,
    )
    row_i = jax.lax.broadcasted_iota(jnp.int32, (K, K), 0)
    col_j = jax.lax.broadcasted_iota(jnp.int32, (K, K), 1)
    lower = (col_j < row_i).astype(jnp.float32)
    L = lower * beta_row * gram
    eye = (col_j == row_i).astype(jnp.float32)
    hp = jax.lax.Precision.HIGHEST
    L2 = jax.lax.dot_general(L, L, (((1,), (0,)), ((), ())),
                             preferred_element_type=jnp.float32, precision=hp)
    L3 = jax.lax.dot_general(L2, L, (((1,), (0,)), ((), ())),
                             preferred_element_type=jnp.float32, precision=hp)
    M = eye - L + L2 - L3

    coeff = jnp.zeros_like(c0)
    for j in range(K):
        coeff = coeff + M[:, j:j + 1][None] * c0[:, j:j + 1, :]

    m = jnp.max(coeff, axis=2, keepdims=True)
    e = jnp.exp((coeff - m) * (1.0 / _TEMP))
    z = jnp.sum(e, axis=2, keepdims=True)
    s_soft = jnp.sum(e * coeff, axis=2, keepdims=True) / z

    iota = jax.lax.broadcasted_iota(jnp.int32, (B, K, T), 2)
    cur = coeff
    s_top = jnp.zeros((B, K, 1), dtype=jnp.float32)
    for _ in range(top_m):
        mx = jnp.max(cur, axis=2, keepdims=True)
        s_top = s_top + mx
        hit = jnp.where(cur == mx, iota, T)
        first = jnp.min(hit, axis=2, keepdims=True)
        cur = jnp.where(iota == first, jnp.float32(-jnp.inf), cur)

    return (0.5 * (s_top + s_soft))[:, :, 0]


def _fused_kernel(e_ref, u_ref, clsw_ref, clsb_ref, beta_ref, alpha_ref,
                  out_ref, coeff_ref, wu_ref, *, B, T, K, top_m):
    i = pl.program_id(0)
    tblks = T // _TB
    a_steps = B * tblks
    u = u_ref[...]

    @pl.when(i < a_steps)
    def _():
        res = jax.lax.dot_general(
            u, e_ref[0], (((1,), (1,)), ((), ())),
            preferred_element_type=jnp.float32,
        )
        for s in range(a_steps):
            @pl.when(i == s)
            def _():
                b, tb = divmod(s, tblks)
                coeff_ref[b, :, tb * _TB:(tb + 1) * _TB] = res

    for s in range(K):
        @pl.when(i == s)
        def _():
            wu_ref[s:s + 1, :] = jax.lax.dot_general(
                u[s:s + 1], clsw_ref[0], (((1,), (1,)), ((), ())),
                preferred_element_type=jnp.float32,
            )

    @pl.when(i == a_steps)
    def _():
        hp = jax.lax.Precision.HIGHEST
        c = _stats_all(u, beta_ref[...], coeff_ref,
                       B=B, K=K, T=T, top_m=top_m)
        ac = alpha_ref[...] * c
        logits = jax.lax.dot_general(
            ac, wu_ref[...], (((1,), (0,)), ((), ())),
            preferred_element_type=jnp.float32, precision=hp,
        )
        bias = jax.lax.dot_general(
            alpha_ref[...], clsb_ref[...], (((1,), (0,)), ((), ())),
            preferred_element_type=jnp.float32, precision=hp,
        )
        out_ref[...] = logits + bias


def kernel(E, v, m_logits, cls_W, cls_b, beta, alpha):
    B, T, D = E.shape
    K = v.shape[0]
    top_m = min(_TOP_M, T)
    tblks = T // _TB
    a_steps = B * tblks

    mk = jax.nn.sigmoid(m_logits)
    vk = v * mk
    U = vk / (jnp.linalg.norm(vk, axis=1, keepdims=True) + _EPS)

    fused = functools.partial(_fused_kernel, B=B, T=T, K=K, top_m=top_m)
    logits = pl.pallas_call(
        fused,
        grid=(a_steps + 1,),
        in_specs=[
            pl.BlockSpec(
                (1, _TB, D),
                lambda i: (jnp.minimum(i, a_steps - 1) // tblks,
                           jnp.minimum(i, a_steps - 1) % tblks, 0)),
            pl.BlockSpec((K, D), lambda i: (0, 0)),
            pl.BlockSpec((1, _NUM_CLASSES, D),
                         lambda i: (jnp.minimum(i, K - 1), 0, 0)),
            pl.BlockSpec((K, _NUM_CLASSES), lambda i: (0, 0)),
            pl.BlockSpec((1, K), lambda i: (0, 0)),
            pl.BlockSpec((1, K), lambda i: (0, 0)),
        ],
        out_specs=pl.BlockSpec((B, _NUM_CLASSES), lambda i: (0, 0)),
        out_shape=jax.ShapeDtypeStruct((B, _NUM_CLASSES), jnp.float32),
        scratch_shapes=[
            pltpu.VMEM((B, K, T), jnp.float32),
            pltpu.VMEM((K, _NUM_CLASSES), jnp.float32),
        ],
    )(E, U, cls_W, cls_b, beta.reshape(1, K), alpha.reshape(1, K))
    return logits


# speedup vs baseline: 1.6284x; 1.0150x over previous
import functools

import jax
import jax.numpy as jnp
from jax.experimental import pallas as pl
from jax.experimental.pallas import tpu as pltpu

_K = 4
_TOP_M = 8
_TEMP = 0.07
_EPS = 1e-06
_NUM_CLASSES = 1000
_TB = 4096


def _stats_for_batch(b, u, beta_row, coeff_ref, c_ref, *, K, T, top_m):
    plane = coeff_ref[b]
    gram = jax.lax.dot_general(
        u, u, (((1,), (1,)), ((), ())),
        preferred_element_type=jnp.float32,
        precision=jax.lax.Precision.HIGHEST,
    )
    row_i = jax.lax.broadcasted_iota(jnp.int32, (K, K), 0)
    col_j = jax.lax.broadcasted_iota(jnp.int32, (K, K), 1)
    lower = (col_j < row_i).astype(jnp.float32)
    L = lower * beta_row * gram
    eye = (col_j == row_i).astype(jnp.float32)
    hp = jax.lax.Precision.HIGHEST
    L2 = jax.lax.dot_general(L, L, (((1,), (0,)), ((), ())),
                             preferred_element_type=jnp.float32, precision=hp)
    L3 = jax.lax.dot_general(L2, L, (((1,), (0,)), ((), ())),
                             preferred_element_type=jnp.float32, precision=hp)
    M = eye - L + L2 - L3

    coeff = jnp.zeros_like(plane)
    for j in range(K):
        coeff = coeff + M[:, j:j + 1] * plane[j:j + 1, :]

    m = jnp.max(coeff, axis=1, keepdims=True)
    e = jnp.exp((coeff - m) * (1.0 / _TEMP))
    z = jnp.sum(e, axis=1, keepdims=True)
    s_soft = jnp.sum(e * coeff, axis=1, keepdims=True) / z

    iota = jax.lax.broadcasted_iota(jnp.int32, (K, T), 1)
    cur = coeff
    s_top = jnp.zeros((K, 1), dtype=jnp.float32)
    for _ in range(top_m):
        mx = jnp.max(cur, axis=1, keepdims=True)
        s_top = s_top + mx
        hit = jnp.where(cur == mx, iota, T)
        first = jnp.min(hit, axis=1, keepdims=True)
        cur = jnp.where(iota == first, jnp.float32(-jnp.inf), cur)

    c_ref[:, b:b + 1] = 0.5 * (s_top + s_soft)


def _fused_kernel(e_ref, u_ref, clsw_ref, clsb_ref, beta_ref, alpha_ref,
                  out_ref, coeff_ref, wu_ref, c_ref, *, B, T, K, top_m):
    i = pl.program_id(0)
    tblks = T // _TB
    a_steps = B * tblks
    u = u_ref[...]

    @pl.when(i < a_steps)
    def _():
        res = jax.lax.dot_general(
            u, e_ref[0], (((1,), (1,)), ((), ())),
            preferred_element_type=jnp.float32,
        )
        for s in range(a_steps):
            @pl.when(i == s)
            def _():
                b, tb = divmod(s, tblks)
                coeff_ref[b, :, tb * _TB:(tb + 1) * _TB] = res

    for s in range(K):
        @pl.when(i == s)
        def _():
            wu_ref[s:s + 1, :] = jax.lax.dot_general(
                u[s:s + 1], clsw_ref[0], (((1,), (1,)), ((), ())),
                preferred_element_type=jnp.float32,
            )

    for b in range(B):
        @pl.when(i == (b + 1) * tblks)
        def _():
            _stats_for_batch(b, u, beta_ref[...], coeff_ref, c_ref,
                             K=K, T=T, top_m=top_m)

    @pl.when(i == a_steps)
    def _():
        hp = jax.lax.Precision.HIGHEST
        ac = alpha_ref[...].reshape(K, 1) * c_ref[...]
        logits = jax.lax.dot_general(
            ac, wu_ref[...], (((0,), (0,)), ((), ())),
            preferred_element_type=jnp.float32, precision=hp,
        )
        bias = jax.lax.dot_general(
            alpha_ref[...], clsb_ref[...], (((1,), (0,)), ((), ())),
            preferred_element_type=jnp.float32, precision=hp,
        )
        out_ref[...] = logits + bias


def kernel(E, v, m_logits, cls_W, cls_b, beta, alpha):
    B, T, D = E.shape
    K = v.shape[0]
    top_m = min(_TOP_M, T)
    tblks = T // _TB
    a_steps = B * tblks

    mk = jax.nn.sigmoid(m_logits)
    vk = v * mk
    U = vk / (jnp.linalg.norm(vk, axis=1, keepdims=True) + _EPS)

    fused = functools.partial(_fused_kernel, B=B, T=T, K=K, top_m=top_m)
    logits = pl.pallas_call(
        fused,
        grid=(a_steps + 1,),
        in_specs=[
            pl.BlockSpec(
                (1, _TB, D),
                lambda i: (jnp.minimum(i, a_steps - 1) // tblks,
                           jnp.minimum(i, a_steps - 1) % tblks, 0)),
            pl.BlockSpec((K, D), lambda i: (0, 0)),
            pl.BlockSpec((1, _NUM_CLASSES, D),
                         lambda i: (jnp.minimum(i, K - 1), 0, 0)),
            pl.BlockSpec((K, _NUM_CLASSES), lambda i: (0, 0)),
            pl.BlockSpec((1, K), lambda i: (0, 0)),
            pl.BlockSpec((1, K), lambda i: (0, 0)),
        ],
        out_specs=pl.BlockSpec((B, _NUM_CLASSES), lambda i: (0, 0)),
        out_shape=jax.ShapeDtypeStruct((B, _NUM_CLASSES), jnp.float32),
        scratch_shapes=[
            pltpu.VMEM((B, K, T), jnp.float32),
            pltpu.VMEM((K, _NUM_CLASSES), jnp.float32),
            pltpu.VMEM((K, B), jnp.float32),
        ],
    )(E, U, cls_W, cls_b, beta.reshape(1, K), alpha.reshape(1, K))
    return logits
